# baseline (speedup 1.0000x reference)
import functools

import jax
import jax.numpy as jnp
from jax import lax
from jax.experimental import pallas as pl
from jax.experimental.pallas import tpu as pltpu
from jax.experimental.pallas import tpu_sc as plsc

_NUM_CLASSES = 1000000
_FEAT = 64
_BATCH = 16384
_LAMBDA_C = 0.001

_NC = 2
_NS = 16
_NW = _NC * _NS
_BPW = _BATCH // _NW
_CH = 128
_NCH = _BPW // _CH
_L = 16


def _partials_kernel(feat_hbm, lab_hbm, cent_hbm, out_hbm,
                     idx_v, rows_v, feat_v, acc_v, sem):
    wid = lax.axis_index("s") * _NC + lax.axis_index("c")

    pltpu.sync_copy(lab_hbm.at[wid], idx_v)
    gathers = [
        pltpu.async_copy(cent_hbm.at[idx_v.at[j]],
                         rows_v.at[pl.ds(j * _CH, _CH)], sem)
        for j in range(_NCH)
    ]
    pltpu.sync_copy(feat_hbm.at[wid], feat_v)
    for g in gathers:
        g.wait()

    def body(r, accs):
        out = []
        for c in range(_FEAT // _L):
            x = feat_v[r, pl.ds(c * _L, _L)]
            y = rows_v[r, pl.ds(c * _L, _L)]
            d = x - y
            out.append(accs[c] + d * d)
        return tuple(out)

    zero = jnp.zeros((_L,), jnp.float32)
    accs = lax.fori_loop(0, _BPW, body, (zero,) * (_FEAT // _L))
    acc_v[...] = accs[0] + accs[1] + accs[2] + accs[3]
    pltpu.sync_copy(acc_v, out_hbm.at[wid])


@functools.partial(
    pl.kernel,
    mesh=plsc.VectorSubcoreMesh(core_axis_name="c", subcore_axis_name="s"),
    out_type=jax.ShapeDtypeStruct((_NW, _L), jnp.float32),
    compiler_params=pltpu.CompilerParams(use_tc_tiling_on_sc=False),
    scratch_types=[
        pltpu.VMEM((_NCH, _CH), jnp.int32),
        pltpu.VMEM((_BPW, _FEAT), jnp.float32),
        pltpu.VMEM((_BPW, _FEAT), jnp.float32),
        pltpu.VMEM((_L,), jnp.float32),
        pltpu.SemaphoreType.DMA,
    ],
)
def _partials(feat_hbm, lab_hbm, cent_hbm, out_hbm,
              idx_v, rows_v, feat_v, acc_v, sem):
    _partials_kernel(feat_hbm, lab_hbm, cent_hbm, out_hbm,
                     idx_v, rows_v, feat_v, acc_v, sem)


def kernel(features, labels, centers):
    feat3 = features.reshape(_NW, _BPW, _FEAT)
    lab3 = labels.astype(jnp.int32).reshape(_NW, _NCH, _CH)
    partials = _partials(feat3, lab3, centers)
    return (_LAMBDA_C * 0.5 / _BATCH) * jnp.sum(partials)
